# baseline (speedup 1.0000x reference)
import functools

import jax
import jax.numpy as jnp
from jax import lax
from jax.experimental import pallas as pl
from jax.experimental.pallas import tpu as pltpu
from jax.experimental.pallas import tpu_sc as plsc

B = 16384
NF = 13
NT = 26
VT = 100000
VS = 100000
LS = 50

_info = plsc.get_sparse_core_info()
NC = _info.num_cores
NS = _info.num_subcores
LANES = _info.num_lanes
NW = NC * NS
CH = B // NW
NJ = CH // LANES

_mesh = plsc.VectorSubcoreMesh(core_axis_name="c", subcore_axis_name="s")

_GROUPS = [(0, 7), (7, 14), (14, 20), (20, 26)]
_WAVES_S = [range(10 * w, 10 * (w + 1)) for w in range(5)]


@functools.partial(
    pl.kernel,
    mesh=_mesh,
    out_type=jax.ShapeDtypeStruct((B,), jnp.float32),
    scratch_types=[
        pltpu.VMEM((LS * CH,), jnp.int32),
        pltpu.VMEM((LS * CH,), jnp.float32),
        pltpu.VMEM((CH,), jnp.float32),
        pltpu.VMEM_SHARED((VS,), jnp.float32),
    ] + [pltpu.SemaphoreType.DMA] * len(_WAVES_S),
)
def _fm_seq(sf_hbm, seq_tab, part_hbm, seq_idx, seq_val, part_v, tab_sh,
            *sems):
    wid = lax.axis_index("s") * NC + lax.axis_index("c")
    base = wid * CH

    @pl.when(lax.axis_index("s") == 0)
    def _():
        pltpu.sync_copy(seq_tab, tab_sh)

    pltpu.sync_copy(sf_hbm.at[wid], seq_idx)
    plsc.subcore_barrier()
    waves = []
    for w, fields in enumerate(_WAVES_S):
        waves.append([
            pltpu.async_copy(tab_sh.at[seq_idx.at[pl.ds(l * CH, CH)]],
                             seq_val.at[pl.ds(l * CH, CH)], sems[w])
            for l in fields])

    for w, fields in enumerate(_WAVES_S):
        for cp in waves[w]:
            cp.wait()

        def acc_body(j, carry, fields=fields, first=(w == 0)):
            js = pl.ds(j * LANES, LANES)
            acc = jnp.zeros((LANES,), jnp.float32) if first else part_v[js]
            for l in fields:
                sl = pl.ds(l * CH + j * LANES, LANES)
                acc = acc + jnp.where(seq_idx[sl] != 0, seq_val[sl], 0.0)
            part_v[js] = acc
            return carry

        lax.fori_loop(0, NJ, acc_body, 0)

    pltpu.sync_copy(part_v, part_hbm.at[pl.ds(base, CH)])


def _make_tok(lo, hi, first, last):
    G = hi - lo
    waves = [range(0, (G + 1) // 2), range((G + 1) // 2, G)]
    scratch = [
        pltpu.VMEM((G * CH,), jnp.int32),
        pltpu.VMEM((G * CH,), jnp.float32),
        pltpu.VMEM((CH,), jnp.float32),
    ]
    if first:
        scratch += [
            pltpu.VMEM((NF * CH,), jnp.float32),
            pltpu.VMEM((NF * LANES,), jnp.float32),
            pltpu.VMEM((LANES,), jnp.float32),
        ]
    if last:
        scratch += [pltpu.VMEM((CH,), jnp.float32)]
    scratch += [pltpu.SemaphoreType.DMA] * len(waves)

    @functools.partial(
        pl.kernel,
        mesh=_mesh,
        out_type=jax.ShapeDtypeStruct((B,), jnp.float32),
        scratch_types=scratch,
    )
    def tok_k(tf_hbm, *rest):
        tabs = rest[:G]
        rest = rest[G:]
        if first:
            ff_hbm, fw_hbm, bias_hbm = rest[:3]
            rest = rest[3:]
        else:
            prev_hbm = rest[0]
            rest = rest[1:]
        if last:
            seqp_hbm = rest[0]
            rest = rest[1:]
        out_hbm = rest[0]
        tok_idx, tok_val, acc_v = rest[1:4]
        rest = rest[4:]
        if first:
            ff_v, fw_v, bias_v = rest[:3]
            rest = rest[3:]
        if last:
            seqp_v = rest[0]
            rest = rest[1:]
        sems = rest

        wid = lax.axis_index("s") * NC + lax.axis_index("c")
        base = wid * CH

        pltpu.sync_copy(tf_hbm.at[wid, pl.ds(lo * CH, G * CH)], tok_idx)
        cps = []
        for w, fields in enumerate(waves):
            cps.append([
                pltpu.async_copy(tabs[t].at[tok_idx.at[pl.ds(t * CH, CH)]],
                                 tok_val.at[pl.ds(t * CH, CH)], sems[w])
                for t in fields])

        if first:
            pltpu.sync_copy(ff_hbm.at[wid], ff_v)
            pltpu.sync_copy(fw_hbm, fw_v)
            pltpu.sync_copy(bias_hbm, bias_v)
        else:
            pltpu.sync_copy(prev_hbm.at[pl.ds(base, CH)], acc_v)
        if last:
            pltpu.sync_copy(seqp_hbm.at[pl.ds(base, CH)], seqp_v)

        if first:
            def base_body(j, carry):
                js = pl.ds(j * LANES, LANES)
                acc = bias_v[pl.ds(0, LANES)]
                for f in range(NF):
                    acc = acc + ff_v[pl.ds(f * CH + j * LANES, LANES)] * fw_v[pl.ds(f * LANES, LANES)]
                acc_v[js] = acc
                return carry

            lax.fori_loop(0, NJ, base_body, 0)

        for w, fields in enumerate(waves):
            for cp in cps[w]:
                cp.wait()
            add_seq = last and w == len(waves) - 1

            def acc_body(j, carry, fields=fields, add_seq=add_seq):
                js = pl.ds(j * LANES, LANES)
                acc = acc_v[js]
                for t in fields:
                    acc = acc + tok_val[pl.ds(t * CH + j * LANES, LANES)]
                if add_seq:
                    acc = acc + seqp_v[js]
                acc_v[js] = acc
                return carry

            lax.fori_loop(0, NJ, acc_body, 0)

        pltpu.sync_copy(acc_v, out_hbm.at[pl.ds(base, CH)])

    return tok_k


_TOK_KERNELS = [
    _make_tok(lo, hi, k == 0, k == len(_GROUPS) - 1)
    for k, (lo, hi) in enumerate(_GROUPS)
]


def _field_major(x, nfields):
    return x.T.reshape(nfields, NW, CH).transpose(1, 0, 2).reshape(NW, nfields * CH)


def kernel(float_fields, token_fields, token_seq_field, float_emb_table,
           token_emb_table, token_seq_emb_table, bias, offsets):
    tf_w = _field_major(token_fields, NT)
    sf_w = _field_major(token_seq_field, LS)
    ff_w = _field_major(float_fields, NF)
    fw_rep = jnp.repeat(float_emb_table.reshape(-1), LANES)
    bias_rep = jnp.broadcast_to(bias.reshape(1), (LANES,))
    del offsets
    part = _fm_seq(sf_w, token_seq_emb_table.reshape(-1))
    tabs = [token_emb_table[i * VT:(i + 1) * VT].reshape(-1)
            for i in range(NT)]
    run = None
    for k, (lo, hi) in enumerate(_GROUPS):
        args = [tf_w] + tabs[lo:hi]
        if k == 0:
            args += [ff_w, fw_rep, bias_rep]
        else:
            args += [run]
        if k == len(_GROUPS) - 1:
            args += [part]
        run = _TOK_KERNELS[k](*args)
    return run.reshape(B, 1)


# speedup vs baseline: 1.2083x; 1.2083x over previous
import functools

import jax
import jax.numpy as jnp
from jax import lax
from jax.experimental import pallas as pl
from jax.experimental.pallas import tpu as pltpu
from jax.experimental.pallas import tpu_sc as plsc

B = 16384
NF = 13
NT = 26
VT = 100000
VS = 100000
LS = 50

_info = plsc.get_sparse_core_info()
NC = _info.num_cores
NS = _info.num_subcores
LANES = _info.num_lanes
NW = NC * NS
CH = B // NW
NJ = CH // LANES

_mesh = plsc.VectorSubcoreMesh(core_axis_name="c", subcore_axis_name="s")

_GROUPS = [(0, 13), (13, 26)]
_WAVES_S = [range(10 * w, 10 * (w + 1)) for w in range(5)]


@functools.partial(
    pl.kernel,
    mesh=_mesh,
    out_type=jax.ShapeDtypeStruct((B,), jnp.float32),
    scratch_types=[
        pltpu.VMEM((LS * CH,), jnp.int32),
        pltpu.VMEM((LS * CH,), jnp.float32),
        pltpu.VMEM((CH,), jnp.float32),
        pltpu.VMEM_SHARED((VS,), jnp.float32),
    ] + [pltpu.SemaphoreType.DMA] * len(_WAVES_S),
)
def _fm_seq(sf_hbm, seq_tab, part_hbm, seq_idx, seq_val, part_v, tab_sh,
            *sems):
    wid = lax.axis_index("s") * NC + lax.axis_index("c")
    base = wid * CH

    @pl.when(lax.axis_index("s") == 0)
    def _():
        pltpu.sync_copy(seq_tab, tab_sh)

    pltpu.sync_copy(sf_hbm.at[wid], seq_idx)
    plsc.subcore_barrier()
    waves = []
    for w, fields in enumerate(_WAVES_S):
        waves.append([
            pltpu.async_copy(tab_sh.at[seq_idx.at[pl.ds(l * CH, CH)]],
                             seq_val.at[pl.ds(l * CH, CH)], sems[w])
            for l in fields])

    for w, fields in enumerate(_WAVES_S):
        for cp in waves[w]:
            cp.wait()

        def acc_body(j, carry, fields=fields, first=(w == 0)):
            js = pl.ds(j * LANES, LANES)
            acc = jnp.zeros((LANES,), jnp.float32) if first else part_v[js]
            for l in fields:
                sl = pl.ds(l * CH + j * LANES, LANES)
                acc = acc + jnp.where(seq_idx[sl] != 0, seq_val[sl], 0.0)
            part_v[js] = acc
            return carry

        lax.fori_loop(0, NJ, acc_body, 0)

    pltpu.sync_copy(part_v, part_hbm.at[pl.ds(base, CH)])


def _make_tok(lo, hi, first, last):
    G = hi - lo
    waves = [range(0, (G + 1) // 2), range((G + 1) // 2, G)]
    scratch = [
        pltpu.VMEM((G * CH,), jnp.int32),
        pltpu.VMEM((G * CH,), jnp.float32),
        pltpu.VMEM((CH,), jnp.float32),
    ]
    if first:
        scratch += [
            pltpu.VMEM((NF * CH,), jnp.float32),
            pltpu.VMEM((NF * LANES,), jnp.float32),
            pltpu.VMEM((LANES,), jnp.float32),
        ]
    if last:
        scratch += [pltpu.VMEM((CH,), jnp.float32)]
    scratch += [pltpu.SemaphoreType.DMA] * len(waves)

    @functools.partial(
        pl.kernel,
        mesh=_mesh,
        out_type=jax.ShapeDtypeStruct((B,), jnp.float32),
        scratch_types=scratch,
    )
    def tok_k(tf_hbm, *rest):
        tabs = rest[:G]
        rest = rest[G:]
        if first:
            ff_hbm, fw_hbm, bias_hbm = rest[:3]
            rest = rest[3:]
        else:
            prev_hbm = rest[0]
            rest = rest[1:]
        if last:
            seqp_hbm = rest[0]
            rest = rest[1:]
        out_hbm = rest[0]
        tok_idx, tok_val, acc_v = rest[1:4]
        rest = rest[4:]
        if first:
            ff_v, fw_v, bias_v = rest[:3]
            rest = rest[3:]
        if last:
            seqp_v = rest[0]
            rest = rest[1:]
        sems = rest

        wid = lax.axis_index("s") * NC + lax.axis_index("c")
        base = wid * CH

        pltpu.sync_copy(tf_hbm.at[wid, pl.ds(lo * CH, G * CH)], tok_idx)
        cps = []
        for w, fields in enumerate(waves):
            cps.append([
                pltpu.async_copy(tabs[t].at[tok_idx.at[pl.ds(t * CH, CH)]],
                                 tok_val.at[pl.ds(t * CH, CH)], sems[w])
                for t in fields])

        if first:
            pltpu.sync_copy(ff_hbm.at[wid], ff_v)
            pltpu.sync_copy(fw_hbm, fw_v)
            pltpu.sync_copy(bias_hbm, bias_v)
        else:
            pltpu.sync_copy(prev_hbm.at[pl.ds(base, CH)], acc_v)
        if last:
            pltpu.sync_copy(seqp_hbm.at[pl.ds(base, CH)], seqp_v)

        if first:
            def base_body(j, carry):
                js = pl.ds(j * LANES, LANES)
                acc = bias_v[pl.ds(0, LANES)]
                for f in range(NF):
                    acc = acc + ff_v[pl.ds(f * CH + j * LANES, LANES)] * fw_v[pl.ds(f * LANES, LANES)]
                acc_v[js] = acc
                return carry

            lax.fori_loop(0, NJ, base_body, 0)

        for w, fields in enumerate(waves):
            for cp in cps[w]:
                cp.wait()
            add_seq = last and w == len(waves) - 1

            def acc_body(j, carry, fields=fields, add_seq=add_seq):
                js = pl.ds(j * LANES, LANES)
                acc = acc_v[js]
                for t in fields:
                    acc = acc + tok_val[pl.ds(t * CH + j * LANES, LANES)]
                if add_seq:
                    acc = acc + seqp_v[js]
                acc_v[js] = acc
                return carry

            lax.fori_loop(0, NJ, acc_body, 0)

        pltpu.sync_copy(acc_v, out_hbm.at[pl.ds(base, CH)])

    return tok_k


_TOK_KERNELS = [
    _make_tok(lo, hi, k == 0, k == len(_GROUPS) - 1)
    for k, (lo, hi) in enumerate(_GROUPS)
]


def _field_major(x, nfields):
    return x.T.reshape(nfields, NW, CH).transpose(1, 0, 2).reshape(NW, nfields * CH)


def kernel(float_fields, token_fields, token_seq_field, float_emb_table,
           token_emb_table, token_seq_emb_table, bias, offsets):
    tf_w = _field_major(token_fields, NT)
    sf_w = _field_major(token_seq_field, LS)
    ff_w = _field_major(float_fields, NF)
    fw_rep = jnp.repeat(float_emb_table.reshape(-1), LANES)
    bias_rep = jnp.broadcast_to(bias.reshape(1), (LANES,))
    del offsets
    part = _fm_seq(sf_w, token_seq_emb_table.reshape(-1))
    tabs = [token_emb_table[i * VT:(i + 1) * VT].reshape(-1)
            for i in range(NT)]
    run = None
    for k, (lo, hi) in enumerate(_GROUPS):
        args = [tf_w] + tabs[lo:hi]
        if k == 0:
            args += [ff_w, fw_rep, bias_rep]
        else:
            args += [run]
        if k == len(_GROUPS) - 1:
            args += [part]
        run = _TOK_KERNELS[k](*args)
    return run.reshape(B, 1)
